# Initial kernel scaffold; baseline (speedup 1.0000x reference)
import jax
import jax.numpy as jnp
from jax.experimental import pallas as pl


def kernel(x, ei0, ei1, ei2, W1_0, b1_0, W1_1, b1_1, W1_2, b1_2, W2_0, b2_0, W2_1, b2_1, W2_2, b2_2):
    raise NotImplementedError("write your pallas kernel here")


# speedup vs baseline: 6.3457x; 6.3457x over previous
import jax
import jax.numpy as jnp
from jax import lax
from jax.experimental import pallas as pl
from jax.experimental.pallas import tpu as pltpu
from jax.experimental.pallas import tpu_sc as plsc

N = 10000
D = 128
E = 320000
NC = 2
NS = 16
NW = NC * NS
EPW = E // NW
CH = 128
KCH = 79
EPWP = KCH * CH
NP = 10240
STRIPE = NP // NS
B = 1024

_mesh = plsc.VectorSubcoreMesh(
    core_axis_name="c", subcore_axis_name="s", num_cores=NC, num_subcores=NS)


def _deg_body(idx_flat, zeros128, ones128, out, idx_c, ones_v, acc):
    c = lax.axis_index("c")
    s = lax.axis_index("s")
    w = c * NS + s
    pltpu.sync_copy(ones128, ones_v)
    for k in range(6):
        pltpu.sync_copy(zeros128, acc.at[pl.ds(s * STRIPE, STRIPE)])
        plsc.subcore_barrier()
        base = (k * NW + w) * EPWP

        def body(j, _, base=base):
            pltpu.sync_copy(idx_flat.at[pl.ds(base + j * CH, CH)], idx_c)
            pltpu.sync_copy(ones_v, acc.at[idx_c], add=True)
            return _

        lax.fori_loop(0, KCH, body, None)
        plsc.subcore_barrier()
        pltpu.sync_copy(acc.at[pl.ds(s * STRIPE, STRIPE)],
                        out.at[pl.ds((k * NC + c) * NP + s * STRIPE, STRIPE)])
        plsc.subcore_barrier()


_deg_call = pl.kernel(
    _deg_body,
    out_type=jax.ShapeDtypeStruct((6 * NC * NP, D), jnp.float32),
    mesh=_mesh,
    scratch_types=[
        pltpu.VMEM((CH,), jnp.int32),
        pltpu.VMEM((CH, D), jnp.float32),
        pltpu.VMEM_SHARED((NP, D), jnp.float32),
    ],
)


def _scat_body(y0, y1, y2, idx_flat, zeros128, S0, S1, S2,
               idx_s, idx_d, buf, acc, sem):
    c = lax.axis_index("c")
    s = lax.axis_index("s")
    w = c * NS + s
    ys = [y0, y1, y2]
    Ss = [S0, S1, S2]
    for r in range(3):
        pltpu.sync_copy(zeros128, acc.at[pl.ds(s * STRIPE, STRIPE)])
        plsc.subcore_barrier()
        sbase = ((2 * r) * NW + w) * EPWP
        dbase = ((2 * r + 1) * NW + w) * EPWP

        def body(j, _, r=r, sbase=sbase, dbase=dbase):
            pltpu.sync_copy(idx_flat.at[pl.ds(sbase + j * CH, CH)], idx_s)
            g = pltpu.async_copy(ys[r].at[idx_s], buf, sem)
            pltpu.sync_copy(idx_flat.at[pl.ds(dbase + j * CH, CH)], idx_d)
            g.wait()
            pltpu.sync_copy(buf, acc.at[idx_d], add=True)
            return _

        lax.fori_loop(0, KCH, body, None)
        plsc.subcore_barrier()
        pltpu.sync_copy(acc.at[pl.ds(s * STRIPE, STRIPE)],
                        Ss[r].at[pl.ds(c * NP + s * STRIPE, STRIPE)])


_scat_call = pl.kernel(
    _scat_body,
    out_type=[jax.ShapeDtypeStruct((NC * NP, D), jnp.float32)] * 3,
    mesh=_mesh,
    scratch_types=[
        pltpu.VMEM((CH,), jnp.int32),
        pltpu.VMEM((CH,), jnp.int32),
        pltpu.VMEM((CH, D), jnp.float32),
        pltpu.VMEM_SHARED((NP, D), jnp.float32),
        pltpu.SemaphoreType.DMA,
    ],
)


def _dense1_body(x_ref, degp_ref, w_ref, y0_ref, y1_ref, y2_ref, n_ref):
    dp = degp_ref[...]
    deg = dp[:, :6] + dp[:, 6:] + 1.0
    nrm = lax.rsqrt(deg)
    n_ref[...] = nrm
    xb = x_ref[...]
    outs = [y0_ref, y1_ref, y2_ref]
    for r in range(3):
        outs[r][...] = jnp.dot(xb * nrm[:, 2 * r:2 * r + 1], w_ref[r])


def _dense1(x_p, degp_n, W1):
    return pl.pallas_call(
        _dense1_body,
        grid=(NP // B,),
        in_specs=[
            pl.BlockSpec((B, D), lambda i: (i, 0)),
            pl.BlockSpec((B, 12), lambda i: (i, 0)),
            pl.BlockSpec((3, D, D), lambda i: (0, 0, 0)),
        ],
        out_specs=[
            pl.BlockSpec((B, D), lambda i: (i, 0)),
            pl.BlockSpec((B, D), lambda i: (i, 0)),
            pl.BlockSpec((B, D), lambda i: (i, 0)),
            pl.BlockSpec((B, 6), lambda i: (i, 0)),
        ],
        out_shape=[jax.ShapeDtypeStruct((NP, D), jnp.float32)] * 3
        + [jax.ShapeDtypeStruct((NP, 6), jnp.float32)],
    )(x_p, degp_n, W1)


def _fuse_body(s0_ref, s1_ref, s2_ref, y0_ref, y1_ref, y2_ref,
               n_ref, b1_ref, w2_ref, o0_ref, o1_ref, o2_ref):
    nrm = n_ref[...]
    bsum = b1_ref[0] + b1_ref[1] + b1_ref[2]
    srefs = [s0_ref, s1_ref, s2_ref]
    yrefs = [y0_ref, y1_ref, y2_ref]
    h = jnp.broadcast_to(bsum[None, :], (B, D))
    for r in range(3):
        tot = srefs[r][0] + srefs[r][1] + yrefs[r][...]
        h = h + tot * nrm[:, 2 * r + 1:2 * r + 2]
    h = jnp.maximum(h, 0.0)
    orefs = [o0_ref, o1_ref, o2_ref]
    for r in range(3):
        orefs[r][...] = jnp.dot(h * nrm[:, 2 * r:2 * r + 1], w2_ref[r])


def _fuse(S0, S1, S2, y0, y1, y2, norms, b1, W2):
    sspec = pl.BlockSpec((2, B, D), lambda i: (0, i, 0))
    yspec = pl.BlockSpec((B, D), lambda i: (i, 0))
    return pl.pallas_call(
        _fuse_body,
        grid=(NP // B,),
        in_specs=[sspec, sspec, sspec, yspec, yspec, yspec,
                  pl.BlockSpec((B, 6), lambda i: (i, 0)),
                  pl.BlockSpec((3, D), lambda i: (0, 0)),
                  pl.BlockSpec((3, D, D), lambda i: (0, 0, 0))],
        out_specs=[yspec, yspec, yspec],
        out_shape=[jax.ShapeDtypeStruct((NP, D), jnp.float32)] * 3,
    )(S0, S1, S2, y0, y1, y2, norms, b1, W2)


def _final_body(s0_ref, s1_ref, s2_ref, y0_ref, y1_ref, y2_ref,
                n_ref, b2_ref, o_ref):
    nrm = n_ref[...]
    bsum = b2_ref[0] + b2_ref[1] + b2_ref[2]
    srefs = [s0_ref, s1_ref, s2_ref]
    yrefs = [y0_ref, y1_ref, y2_ref]
    o = jnp.broadcast_to(bsum[None, :], (B, D))
    for r in range(3):
        tot = srefs[r][0] + srefs[r][1] + yrefs[r][...]
        o = o + tot * nrm[:, 2 * r + 1:2 * r + 2]
    o_ref[...] = o


def _final(S0, S1, S2, y0, y1, y2, norms, b2):
    sspec = pl.BlockSpec((2, B, D), lambda i: (0, i, 0))
    yspec = pl.BlockSpec((B, D), lambda i: (i, 0))
    return pl.pallas_call(
        _final_body,
        grid=(NP // B,),
        in_specs=[sspec, sspec, sspec, yspec, yspec, yspec,
                  pl.BlockSpec((B, 6), lambda i: (i, 0)),
                  pl.BlockSpec((3, D), lambda i: (0, 0))],
        out_specs=yspec,
        out_shape=jax.ShapeDtypeStruct((NP, D), jnp.float32),
    )(S0, S1, S2, y0, y1, y2, norms, b2)


def kernel(x, ei0, ei1, ei2,
           W1_0, b1_0, W1_1, b1_1, W1_2, b1_2,
           W2_0, b2_0, W2_1, b2_1, W2_2, b2_2):
    pad = jnp.broadcast_to(
        N + (jnp.arange(EPWP - EPW, dtype=jnp.int32) % (NP - N)),
        (6 * NW, EPWP - EPW))
    eis = jnp.concatenate([ei0, ei1, ei2], axis=0).reshape(6 * NW, EPW)
    idx_flat = jnp.concatenate([eis, pad], axis=1).reshape(-1)

    zeros128 = jnp.zeros((STRIPE, D), jnp.float32)
    ones128 = jnp.ones((CH, D), jnp.float32)
    x_p = jnp.concatenate([x, jnp.zeros((NP - N, D), x.dtype)], axis=0)
    W1 = jnp.stack([W1_0, W1_1, W1_2])
    W2 = jnp.stack([W2_0, W2_1, W2_2])
    b1 = jnp.stack([b1_0, b1_1, b1_2])
    b2 = jnp.stack([b2_0, b2_1, b2_2])

    degp = _deg_call(idx_flat, zeros128, ones128)
    degp_n = degp[:, 0].reshape(6, NC, NP).transpose(2, 1, 0).reshape(NP, 12)
    y10, y11, y12, norms = _dense1(x_p, degp_n, W1)
    S10, S11, S12 = _scat_call(y10, y11, y12, idx_flat, zeros128)
    rs = lambda S: S.reshape(NC, NP, D)
    y20, y21, y22 = _fuse(rs(S10), rs(S11), rs(S12), y10, y11, y12,
                          norms, b1, W2)
    S20, S21, S22 = _scat_call(y20, y21, y22, idx_flat, zeros128)
    out = _final(rs(S20), rs(S21), rs(S22), y20, y21, y22, norms, b2)
    return out[:N]


# speedup vs baseline: 7.8217x; 1.2326x over previous
import jax
import jax.numpy as jnp
from jax import lax
from jax.experimental import pallas as pl
from jax.experimental.pallas import tpu as pltpu
from jax.experimental.pallas import tpu_sc as plsc

N = 10000
D = 128
E = 320000
NC = 2
NS = 16
NW = NC * NS
EPW = E // NW
CH = 128
KCH = 79
EPWP = KCH * CH
NP = 10240
STRIPE = NP // NS
B = 1024

_mesh = plsc.VectorSubcoreMesh(
    core_axis_name="c", subcore_axis_name="s", num_cores=NC, num_subcores=NS)


def _deg_body(idx_flat, zeros128, ones128, out, idx_c, ones_v, acc):
    c = lax.axis_index("c")
    s = lax.axis_index("s")
    w = c * NS + s
    pltpu.sync_copy(ones128, ones_v)
    for k in range(6):
        pltpu.sync_copy(zeros128, acc.at[pl.ds(s * STRIPE, STRIPE)])
        plsc.subcore_barrier()
        base = (k * NW + w) * EPWP

        def body(j, _, base=base):
            pltpu.sync_copy(idx_flat.at[pl.ds(base + j * CH, CH)], idx_c)
            pltpu.sync_copy(ones_v, acc.at[idx_c], add=True)
            return _

        lax.fori_loop(0, KCH, body, None)
        plsc.subcore_barrier()
        pltpu.sync_copy(acc.at[pl.ds(s * STRIPE, STRIPE)],
                        out.at[pl.ds((k * NC + c) * NP + s * STRIPE, STRIPE)])
        plsc.subcore_barrier()


_deg_call = pl.kernel(
    _deg_body,
    out_type=jax.ShapeDtypeStruct((6 * NC * NP, D), jnp.float32),
    mesh=_mesh,
    scratch_types=[
        pltpu.VMEM((CH,), jnp.int32),
        pltpu.VMEM((CH, D), jnp.float32),
        pltpu.VMEM_SHARED((NP, D), jnp.float32),
    ],
)


def _scat_body(y0, y1, y2, idx_flat, zeros128, S0, S1, S2,
               idx_s0, idx_s1, idx_d0, idx_d1, buf0, buf1, acc, g0, g1):
    c = lax.axis_index("c")
    s = lax.axis_index("s")
    w = c * NS + s
    ys = [y0, y1, y2]
    Ss = [S0, S1, S2]
    par = [(idx_s0, idx_d0, buf0, g0), (idx_s1, idx_d1, buf1, g1)]
    for r in range(3):
        pltpu.sync_copy(zeros128, acc.at[pl.ds(s * STRIPE, STRIPE)])
        plsc.subcore_barrier()
        sbase = ((2 * r) * NW + w) * EPWP
        dbase = ((2 * r + 1) * NW + w) * EPWP

        def stage(j, p, r=r, sbase=sbase, dbase=dbase):
            i_s, i_d, bf, g = par[p]
            pltpu.sync_copy(idx_flat.at[pl.ds(sbase + j * CH, CH)], i_s)
            pltpu.async_copy(ys[r].at[i_s], bf, g)
            pltpu.sync_copy(idx_flat.at[pl.ds(dbase + j * CH, CH)], i_d)

        def finish(j, p, r=r):
            i_s, i_d, bf, g = par[p]
            pltpu.make_async_copy(ys[r].at[i_s], bf, g).wait()
            pltpu.sync_copy(bf, acc.at[i_d], add=True)

        stage(0, 0)

        def body(p, _):
            stage(2 * p + 1, 1)
            finish(2 * p, 0)
            stage(2 * p + 2, 0)
            finish(2 * p + 1, 1)
            return _

        lax.fori_loop(0, (KCH - 1) // 2, body, None)
        finish(KCH - 1, 0)
        plsc.subcore_barrier()
        pltpu.sync_copy(acc.at[pl.ds(s * STRIPE, STRIPE)],
                        Ss[r].at[pl.ds(c * NP + s * STRIPE, STRIPE)])


_scat_call = pl.kernel(
    _scat_body,
    out_type=[jax.ShapeDtypeStruct((NC * NP, D), jnp.float32)] * 3,
    mesh=_mesh,
    scratch_types=[
        pltpu.VMEM((CH,), jnp.int32),
        pltpu.VMEM((CH,), jnp.int32),
        pltpu.VMEM((CH,), jnp.int32),
        pltpu.VMEM((CH,), jnp.int32),
        pltpu.VMEM((CH, D), jnp.float32),
        pltpu.VMEM((CH, D), jnp.float32),
        pltpu.VMEM_SHARED((NP, D), jnp.float32),
        pltpu.SemaphoreType.DMA,
        pltpu.SemaphoreType.DMA,
    ],
)


def _dense1_body(x_ref, degp_ref, w_ref, y0_ref, y1_ref, y2_ref, n_ref):
    dp = degp_ref[...]
    deg = dp[:, :6] + dp[:, 6:] + 1.0
    nrm = lax.rsqrt(deg)
    n_ref[...] = nrm
    xb = x_ref[...]
    outs = [y0_ref, y1_ref, y2_ref]
    for r in range(3):
        outs[r][...] = jnp.dot(xb * nrm[:, 2 * r:2 * r + 1], w_ref[r])


def _dense1(x_p, degp_n, W1):
    return pl.pallas_call(
        _dense1_body,
        grid=(NP // B,),
        in_specs=[
            pl.BlockSpec((B, D), lambda i: (i, 0)),
            pl.BlockSpec((B, 12), lambda i: (i, 0)),
            pl.BlockSpec((3, D, D), lambda i: (0, 0, 0)),
        ],
        out_specs=[
            pl.BlockSpec((B, D), lambda i: (i, 0)),
            pl.BlockSpec((B, D), lambda i: (i, 0)),
            pl.BlockSpec((B, D), lambda i: (i, 0)),
            pl.BlockSpec((B, 6), lambda i: (i, 0)),
        ],
        out_shape=[jax.ShapeDtypeStruct((NP, D), jnp.float32)] * 3
        + [jax.ShapeDtypeStruct((NP, 6), jnp.float32)],
    )(x_p, degp_n, W1)


def _fuse_body(s0_ref, s1_ref, s2_ref, y0_ref, y1_ref, y2_ref,
               n_ref, b1_ref, w2_ref, o0_ref, o1_ref, o2_ref):
    nrm = n_ref[...]
    bsum = b1_ref[0] + b1_ref[1] + b1_ref[2]
    srefs = [s0_ref, s1_ref, s2_ref]
    yrefs = [y0_ref, y1_ref, y2_ref]
    h = jnp.broadcast_to(bsum[None, :], (B, D))
    for r in range(3):
        tot = srefs[r][0] + srefs[r][1] + yrefs[r][...]
        h = h + tot * nrm[:, 2 * r + 1:2 * r + 2]
    h = jnp.maximum(h, 0.0)
    orefs = [o0_ref, o1_ref, o2_ref]
    for r in range(3):
        orefs[r][...] = jnp.dot(h * nrm[:, 2 * r:2 * r + 1], w2_ref[r])


def _fuse(S0, S1, S2, y0, y1, y2, norms, b1, W2):
    sspec = pl.BlockSpec((2, B, D), lambda i: (0, i, 0))
    yspec = pl.BlockSpec((B, D), lambda i: (i, 0))
    return pl.pallas_call(
        _fuse_body,
        grid=(NP // B,),
        in_specs=[sspec, sspec, sspec, yspec, yspec, yspec,
                  pl.BlockSpec((B, 6), lambda i: (i, 0)),
                  pl.BlockSpec((3, D), lambda i: (0, 0)),
                  pl.BlockSpec((3, D, D), lambda i: (0, 0, 0))],
        out_specs=[yspec, yspec, yspec],
        out_shape=[jax.ShapeDtypeStruct((NP, D), jnp.float32)] * 3,
    )(S0, S1, S2, y0, y1, y2, norms, b1, W2)


def _final_body(s0_ref, s1_ref, s2_ref, y0_ref, y1_ref, y2_ref,
                n_ref, b2_ref, o_ref):
    nrm = n_ref[...]
    bsum = b2_ref[0] + b2_ref[1] + b2_ref[2]
    srefs = [s0_ref, s1_ref, s2_ref]
    yrefs = [y0_ref, y1_ref, y2_ref]
    o = jnp.broadcast_to(bsum[None, :], (B, D))
    for r in range(3):
        tot = srefs[r][0] + srefs[r][1] + yrefs[r][...]
        o = o + tot * nrm[:, 2 * r + 1:2 * r + 2]
    o_ref[...] = o


def _final(S0, S1, S2, y0, y1, y2, norms, b2):
    sspec = pl.BlockSpec((2, B, D), lambda i: (0, i, 0))
    yspec = pl.BlockSpec((B, D), lambda i: (i, 0))
    return pl.pallas_call(
        _final_body,
        grid=(NP // B,),
        in_specs=[sspec, sspec, sspec, yspec, yspec, yspec,
                  pl.BlockSpec((B, 6), lambda i: (i, 0)),
                  pl.BlockSpec((3, D), lambda i: (0, 0))],
        out_specs=yspec,
        out_shape=jax.ShapeDtypeStruct((NP, D), jnp.float32),
    )(S0, S1, S2, y0, y1, y2, norms, b2)


def kernel(x, ei0, ei1, ei2,
           W1_0, b1_0, W1_1, b1_1, W1_2, b1_2,
           W2_0, b2_0, W2_1, b2_1, W2_2, b2_2):
    pad = jnp.broadcast_to(
        N + (jnp.arange(EPWP - EPW, dtype=jnp.int32) % (NP - N)),
        (6 * NW, EPWP - EPW))
    eis = jnp.concatenate([ei0, ei1, ei2], axis=0).reshape(6 * NW, EPW)
    idx_flat = jnp.concatenate([eis, pad], axis=1).reshape(-1)

    zeros128 = jnp.zeros((STRIPE, D), jnp.float32)
    ones128 = jnp.ones((CH, D), jnp.float32)
    x_p = jnp.concatenate([x, jnp.zeros((NP - N, D), x.dtype)], axis=0)
    W1 = jnp.stack([W1_0, W1_1, W1_2])
    W2 = jnp.stack([W2_0, W2_1, W2_2])
    b1 = jnp.stack([b1_0, b1_1, b1_2])
    b2 = jnp.stack([b2_0, b2_1, b2_2])

    degp = _deg_call(idx_flat, zeros128, ones128)
    degp_n = degp[:, 0].reshape(6, NC, NP).transpose(2, 1, 0).reshape(NP, 12)
    y10, y11, y12, norms = _dense1(x_p, degp_n, W1)
    S10, S11, S12 = _scat_call(y10, y11, y12, idx_flat, zeros128)
    rs = lambda S: S.reshape(NC, NP, D)
    y20, y21, y22 = _fuse(rs(S10), rs(S11), rs(S12), y10, y11, y12,
                          norms, b1, W2)
    S20, S21, S22 = _scat_call(y20, y21, y22, idx_flat, zeros128)
    out = _final(rs(S20), rs(S21), rs(S22), y20, y21, y22, norms, b2)
    return out[:N]


# speedup vs baseline: 8.8781x; 1.1351x over previous
import jax
import jax.numpy as jnp
from jax import lax
from jax.experimental import pallas as pl
from jax.experimental.pallas import tpu as pltpu
from jax.experimental.pallas import tpu_sc as plsc

N = 10000
D = 128
E = 320000
NC = 2
NS = 16
NW = NC * NS
EPW = E // NW
CH = 128
KCH = 79
EPWP = KCH * CH
NP = 10240
STRIPE = NP // NS
B = 1024

_mesh = plsc.VectorSubcoreMesh(
    core_axis_name="c", subcore_axis_name="s", num_cores=NC, num_subcores=NS)


def _deg_body(idx_flat, zeros128, ones128, out, ic0, ic1, ones_v, acc, a0, a1):
    c = lax.axis_index("c")
    s = lax.axis_index("s")
    w = c * NS + s
    pltpu.sync_copy(ones128, ones_v)
    par = [(ic0, a0), (ic1, a1)]
    for k in range(6):
        pltpu.sync_copy(zeros128, acc.at[pl.ds(s * STRIPE, STRIPE)])
        plsc.subcore_barrier()
        base = (k * NW + w) * EPWP

        def load(j, p, base=base):
            ic, a = par[p]
            pltpu.async_copy(idx_flat.at[pl.ds(base + j * CH, CH)], ic, a)

        def scat(j, p, base=base):
            ic, a = par[p]
            pltpu.make_async_copy(
                idx_flat.at[pl.ds(base + j * CH, CH)], ic, a).wait()
            pltpu.sync_copy(ones_v, acc.at[ic], add=True)

        load(0, 0)

        def body(p, _):
            load(2 * p + 1, 1)
            scat(2 * p, 0)
            load(2 * p + 2, 0)
            scat(2 * p + 1, 1)
            return _

        lax.fori_loop(0, (KCH - 1) // 2, body, None)
        scat(KCH - 1, 0)
        plsc.subcore_barrier()
        pltpu.sync_copy(acc.at[pl.ds(s * STRIPE, STRIPE)],
                        out.at[pl.ds((k * NC + c) * NP + s * STRIPE, STRIPE)])
        plsc.subcore_barrier()


_deg_call = pl.kernel(
    _deg_body,
    out_type=jax.ShapeDtypeStruct((6 * NC * NP, D), jnp.float32),
    mesh=_mesh,
    scratch_types=[
        pltpu.VMEM((CH,), jnp.int32),
        pltpu.VMEM((CH,), jnp.int32),
        pltpu.VMEM((CH, D), jnp.float32),
        pltpu.VMEM_SHARED((NP, D), jnp.float32),
        pltpu.SemaphoreType.DMA,
        pltpu.SemaphoreType.DMA,
    ],
)


def _scat_body(y0, y1, y2, idx_flat, zeros128, S0, S1, S2,
               idx_s0, idx_s1, idx_d0, idx_d1, buf0, buf1, acc, g0, g1):
    c = lax.axis_index("c")
    s = lax.axis_index("s")
    w = c * NS + s
    ys = [y0, y1, y2]
    Ss = [S0, S1, S2]
    par = [(idx_s0, idx_d0, buf0, g0), (idx_s1, idx_d1, buf1, g1)]
    for r in range(3):
        pltpu.sync_copy(zeros128, acc.at[pl.ds(s * STRIPE, STRIPE)])
        plsc.subcore_barrier()
        sbase = ((2 * r) * NW + w) * EPWP
        dbase = ((2 * r + 1) * NW + w) * EPWP

        def stage(j, p, r=r, sbase=sbase, dbase=dbase):
            i_s, i_d, bf, g = par[p]
            pltpu.sync_copy(idx_flat.at[pl.ds(sbase + j * CH, CH)], i_s)
            pltpu.async_copy(ys[r].at[i_s], bf, g)
            pltpu.sync_copy(idx_flat.at[pl.ds(dbase + j * CH, CH)], i_d)

        def finish(j, p, r=r):
            i_s, i_d, bf, g = par[p]
            pltpu.make_async_copy(ys[r].at[i_s], bf, g).wait()
            pltpu.sync_copy(bf, acc.at[i_d], add=True)

        stage(0, 0)

        def body(p, _):
            stage(2 * p + 1, 1)
            finish(2 * p, 0)
            stage(2 * p + 2, 0)
            finish(2 * p + 1, 1)
            return _

        lax.fori_loop(0, (KCH - 1) // 2, body, None)
        finish(KCH - 1, 0)
        plsc.subcore_barrier()
        pltpu.sync_copy(acc.at[pl.ds(s * STRIPE, STRIPE)],
                        Ss[r].at[pl.ds(c * NP + s * STRIPE, STRIPE)])


_scat_call = pl.kernel(
    _scat_body,
    out_type=[jax.ShapeDtypeStruct((NC * NP, D), jnp.float32)] * 3,
    mesh=_mesh,
    scratch_types=[
        pltpu.VMEM((CH,), jnp.int32),
        pltpu.VMEM((CH,), jnp.int32),
        pltpu.VMEM((CH,), jnp.int32),
        pltpu.VMEM((CH,), jnp.int32),
        pltpu.VMEM((CH, D), jnp.float32),
        pltpu.VMEM((CH, D), jnp.float32),
        pltpu.VMEM_SHARED((NP, D), jnp.float32),
        pltpu.SemaphoreType.DMA,
        pltpu.SemaphoreType.DMA,
    ],
)


def _dense1_body(x_ref, degp_ref, w_ref, y0_ref, y1_ref, y2_ref, n_ref):
    dp = degp_ref[...]
    deg = dp[:, :6] + dp[:, 6:] + 1.0
    nrm = lax.rsqrt(deg)
    n_ref[...] = nrm
    xb = x_ref[...]
    outs = [y0_ref, y1_ref, y2_ref]
    for r in range(3):
        outs[r][...] = jnp.dot(xb * nrm[:, 2 * r:2 * r + 1], w_ref[r])


def _dense1(x_p, degp_n, W1):
    return pl.pallas_call(
        _dense1_body,
        grid=(NP // B,),
        in_specs=[
            pl.BlockSpec((B, D), lambda i: (i, 0)),
            pl.BlockSpec((B, 12), lambda i: (i, 0)),
            pl.BlockSpec((3, D, D), lambda i: (0, 0, 0)),
        ],
        out_specs=[
            pl.BlockSpec((B, D), lambda i: (i, 0)),
            pl.BlockSpec((B, D), lambda i: (i, 0)),
            pl.BlockSpec((B, D), lambda i: (i, 0)),
            pl.BlockSpec((B, 6), lambda i: (i, 0)),
        ],
        out_shape=[jax.ShapeDtypeStruct((NP, D), jnp.float32)] * 3
        + [jax.ShapeDtypeStruct((NP, 6), jnp.float32)],
    )(x_p, degp_n, W1)


def _fuse_body(s0_ref, s1_ref, s2_ref, y0_ref, y1_ref, y2_ref,
               n_ref, b1_ref, w2_ref, o0_ref, o1_ref, o2_ref):
    nrm = n_ref[...]
    bsum = b1_ref[0] + b1_ref[1] + b1_ref[2]
    srefs = [s0_ref, s1_ref, s2_ref]
    yrefs = [y0_ref, y1_ref, y2_ref]
    h = jnp.broadcast_to(bsum[None, :], (B, D))
    for r in range(3):
        tot = srefs[r][0] + srefs[r][1] + yrefs[r][...]
        h = h + tot * nrm[:, 2 * r + 1:2 * r + 2]
    h = jnp.maximum(h, 0.0)
    orefs = [o0_ref, o1_ref, o2_ref]
    for r in range(3):
        orefs[r][...] = jnp.dot(h * nrm[:, 2 * r:2 * r + 1], w2_ref[r])


def _fuse(S0, S1, S2, y0, y1, y2, norms, b1, W2):
    sspec = pl.BlockSpec((2, B, D), lambda i: (0, i, 0))
    yspec = pl.BlockSpec((B, D), lambda i: (i, 0))
    return pl.pallas_call(
        _fuse_body,
        grid=(NP // B,),
        in_specs=[sspec, sspec, sspec, yspec, yspec, yspec,
                  pl.BlockSpec((B, 6), lambda i: (i, 0)),
                  pl.BlockSpec((3, D), lambda i: (0, 0)),
                  pl.BlockSpec((3, D, D), lambda i: (0, 0, 0))],
        out_specs=[yspec, yspec, yspec],
        out_shape=[jax.ShapeDtypeStruct((NP, D), jnp.float32)] * 3,
    )(S0, S1, S2, y0, y1, y2, norms, b1, W2)


def _final_body(s0_ref, s1_ref, s2_ref, y0_ref, y1_ref, y2_ref,
                n_ref, b2_ref, o_ref):
    nrm = n_ref[...]
    bsum = b2_ref[0] + b2_ref[1] + b2_ref[2]
    srefs = [s0_ref, s1_ref, s2_ref]
    yrefs = [y0_ref, y1_ref, y2_ref]
    o = jnp.broadcast_to(bsum[None, :], (B, D))
    for r in range(3):
        tot = srefs[r][0] + srefs[r][1] + yrefs[r][...]
        o = o + tot * nrm[:, 2 * r + 1:2 * r + 2]
    o_ref[...] = o


def _final(S0, S1, S2, y0, y1, y2, norms, b2):
    sspec = pl.BlockSpec((2, B, D), lambda i: (0, i, 0))
    yspec = pl.BlockSpec((B, D), lambda i: (i, 0))
    return pl.pallas_call(
        _final_body,
        grid=(NP // B,),
        in_specs=[sspec, sspec, sspec, yspec, yspec, yspec,
                  pl.BlockSpec((B, 6), lambda i: (i, 0)),
                  pl.BlockSpec((3, D), lambda i: (0, 0))],
        out_specs=yspec,
        out_shape=jax.ShapeDtypeStruct((NP, D), jnp.float32),
    )(S0, S1, S2, y0, y1, y2, norms, b2)


def kernel(x, ei0, ei1, ei2,
           W1_0, b1_0, W1_1, b1_1, W1_2, b1_2,
           W2_0, b2_0, W2_1, b2_1, W2_2, b2_2):
    pad = jnp.broadcast_to(
        N + (jnp.arange(EPWP - EPW, dtype=jnp.int32) % (NP - N)),
        (6 * NW, EPWP - EPW))
    eis = jnp.concatenate([ei0, ei1, ei2], axis=0).reshape(6 * NW, EPW)
    idx_flat = jnp.concatenate([eis, pad], axis=1).reshape(-1)

    zeros128 = jnp.zeros((STRIPE, D), jnp.float32)
    ones128 = jnp.ones((CH, D), jnp.float32)
    x_p = jnp.concatenate([x, jnp.zeros((NP - N, D), x.dtype)], axis=0)
    W1 = jnp.stack([W1_0, W1_1, W1_2])
    W2 = jnp.stack([W2_0, W2_1, W2_2])
    b1 = jnp.stack([b1_0, b1_1, b1_2])
    b2 = jnp.stack([b2_0, b2_1, b2_2])

    degp = _deg_call(idx_flat, zeros128, ones128)
    degp_n = degp[:, 0].reshape(6, NC, NP).transpose(2, 1, 0).reshape(NP, 12)
    y10, y11, y12, norms = _dense1(x_p, degp_n, W1)
    S10, S11, S12 = _scat_call(y10, y11, y12, idx_flat, zeros128)
    rs = lambda S: S.reshape(NC, NP, D)
    y20, y21, y22 = _fuse(rs(S10), rs(S11), rs(S12), y10, y11, y12,
                          norms, b1, W2)
    S20, S21, S22 = _scat_call(y20, y21, y22, idx_flat, zeros128)
    out = _final(rs(S20), rs(S21), rs(S22), y20, y21, y22, norms, b2)
    return out[:N]
